# baseline (speedup 1.0000x reference)
import functools

import jax
import jax.numpy as jnp
from jax.experimental import pallas as pl
from jax.experimental.pallas import tpu as pltpu


def _ldr_kernel(idx_ref, h0_ref, a2_ref, b2_ref, alog_ref, dtb_ref,
                qT_ref, kT_ref, v_ref, o_ref, *, T):
    HV, K, V = h0_ref.shape[1], h0_ref.shape[2], h0_ref.shape[3]
    h = h0_ref[0]
    x = a2_ref[0] + dtb_ref[:]
    sp = jnp.where(x <= 20.0, jnp.log1p(jnp.exp(jnp.minimum(x, 20.0))), x)
    gam = jnp.exp(-jnp.exp(alog_ref[:]) * sp)
    beta = jax.nn.sigmoid(b2_ref[0])
    gamB = jnp.broadcast_to(gam[:, :, None], (HV, T, V))
    betaB = jnp.broadcast_to(beta[:, :, None], (HV, T, V))
    qT = qT_ref[0]
    kT = kT_ref[0]

    def bf(z):
        return z.astype(jnp.bfloat16).astype(jnp.float32)

    for t in range(T):
        h = h * gamB[:, t:t + 1, :]
        kcol = kT[:, :, t:t + 1]
        kv = jnp.sum(bf(kcol) * bf(h), axis=1)
        vres = (v_ref[0, t] - kv) * betaB[:, t, :]
        h = h + kcol * vres[:, None, :]
        o_ref[0, t] = jnp.sum(bf(qT[:, :, t:t + 1]) * bf(h), axis=1)


def kernel(A_log, a, dt_bias, q, k, v, b, initial_state_source, initial_state_indices):
    B, T, H, K = q.shape
    HV, V = v.shape[2], v.shape[3]
    rep = HV // H
    scale = K ** (-0.5)

    q_f = q.astype(jnp.float32)
    k_f = k.astype(jnp.float32)
    qT = jnp.repeat(q_f * scale, rep, axis=2).transpose(0, 2, 3, 1)
    kT = jnp.repeat(k_f, rep, axis=2).transpose(0, 2, 3, 1)
    v2 = v.astype(jnp.float32)
    a2 = a.astype(jnp.float32).reshape(B, T, HV).transpose(0, 2, 1)
    b2 = b.astype(jnp.float32).reshape(B, T, HV).transpose(0, 2, 1)
    alog = A_log.astype(jnp.float32).reshape(HV, 1)
    dtb = dt_bias.astype(jnp.float32).reshape(HV, 1)
    src = initial_state_source.astype(jnp.float32)

    grid_spec = pltpu.PrefetchScalarGridSpec(
        num_scalar_prefetch=1,
        grid=(B,),
        in_specs=[
            pl.BlockSpec((1, HV, K, V), lambda i, idx: (idx[i], 0, 0, 0)),
            pl.BlockSpec((1, HV, T), lambda i, idx: (i, 0, 0)),
            pl.BlockSpec((1, HV, T), lambda i, idx: (i, 0, 0)),
            pl.BlockSpec((HV, 1), lambda i, idx: (0, 0)),
            pl.BlockSpec((HV, 1), lambda i, idx: (0, 0)),
            pl.BlockSpec((1, HV, K, T), lambda i, idx: (i, 0, 0, 0)),
            pl.BlockSpec((1, HV, K, T), lambda i, idx: (i, 0, 0, 0)),
            pl.BlockSpec((1, T, HV, V), lambda i, idx: (i, 0, 0, 0)),
        ],
        out_specs=pl.BlockSpec((1, T, HV, V), lambda i, idx: (i, 0, 0, 0)),
    )
    body = functools.partial(_ldr_kernel, T=T)
    o = pl.pallas_call(
        body,
        grid_spec=grid_spec,
        out_shape=jax.ShapeDtypeStruct((B, T, HV, V), jnp.float32),
    )(initial_state_indices, src, a2, b2, alog, dtb, qT, kT, v2)
    return o.astype(v.dtype)


# speedup vs baseline: 1.5797x; 1.5797x over previous
import functools

import jax
import jax.numpy as jnp
from jax.experimental import pallas as pl
from jax.experimental.pallas import tpu as pltpu


def _ldr_kernel(idx_ref, h0_ref, a2_ref, b2_ref, alog_ref, dtb_ref,
                qT_ref, kT_ref, v_ref, o_ref, *, T, REP):
    HV, K, V = h0_ref.shape[1], h0_ref.shape[2], h0_ref.shape[3]
    H = HV // REP
    h = h0_ref[0].reshape(H, REP, K, V)
    x = a2_ref[0] + dtb_ref[:]
    sp = jnp.where(x <= 20.0, jnp.log1p(jnp.exp(jnp.minimum(x, 20.0))), x)
    gam = jnp.exp(-jnp.exp(alog_ref[:]) * sp)
    beta = jax.nn.sigmoid(b2_ref[0])
    gamB = jnp.broadcast_to(gam[:, :, None], (HV, T, V)).reshape(H, REP, T, V)
    betaB = jnp.broadcast_to(beta[:, :, None], (HV, T, V))
    qT = qT_ref[0]
    kT = kT_ref[0]

    def bf(z):
        return z.astype(jnp.bfloat16).astype(jnp.float32)

    for t in range(T):
        h = h * gamB[:, :, t:t + 1, :]
        kcol = kT[:, None, :, t:t + 1]
        kv = jnp.sum(bf(kcol) * bf(h), axis=2)
        kv = kv.reshape(HV, V)
        vres = (v_ref[0, t] - kv) * betaB[:, t, :]
        vres4 = vres.reshape(H, REP, 1, V)
        h = h + kcol * vres4
        o_t = jnp.sum(bf(qT[:, None, :, t:t + 1]) * bf(h), axis=2)
        o_ref[0, t] = o_t.reshape(HV, V)


def kernel(A_log, a, dt_bias, q, k, v, b, initial_state_source, initial_state_indices):
    B, T, H, K = q.shape
    HV, V = v.shape[2], v.shape[3]
    rep = HV // H
    scale = K ** (-0.5)

    q_f = q.astype(jnp.float32)
    k_f = k.astype(jnp.float32)
    qT = (q_f * scale).transpose(0, 2, 3, 1)
    kT = k_f.transpose(0, 2, 3, 1)
    v2 = v.astype(jnp.float32)
    a2 = a.astype(jnp.float32).reshape(B, T, HV).transpose(0, 2, 1)
    b2 = b.astype(jnp.float32).reshape(B, T, HV).transpose(0, 2, 1)
    alog = A_log.astype(jnp.float32).reshape(HV, 1)
    dtb = dt_bias.astype(jnp.float32).reshape(HV, 1)
    src = initial_state_source.astype(jnp.float32)

    grid_spec = pltpu.PrefetchScalarGridSpec(
        num_scalar_prefetch=1,
        grid=(B,),
        in_specs=[
            pl.BlockSpec((1, HV, K, V), lambda i, idx: (idx[i], 0, 0, 0)),
            pl.BlockSpec((1, HV, T), lambda i, idx: (i, 0, 0)),
            pl.BlockSpec((1, HV, T), lambda i, idx: (i, 0, 0)),
            pl.BlockSpec((HV, 1), lambda i, idx: (0, 0)),
            pl.BlockSpec((HV, 1), lambda i, idx: (0, 0)),
            pl.BlockSpec((1, H, K, T), lambda i, idx: (i, 0, 0, 0)),
            pl.BlockSpec((1, H, K, T), lambda i, idx: (i, 0, 0, 0)),
            pl.BlockSpec((1, T, HV, V), lambda i, idx: (i, 0, 0, 0)),
        ],
        out_specs=pl.BlockSpec((1, T, HV, V), lambda i, idx: (i, 0, 0, 0)),
    )
    body = functools.partial(_ldr_kernel, T=T, REP=rep)
    o = pl.pallas_call(
        body,
        grid_spec=grid_spec,
        out_shape=jax.ShapeDtypeStruct((B, T, HV, V), jnp.float32),
    )(initial_state_indices, src, a2, b2, alog, dtb, qT, kT, v2)
    return o.astype(v.dtype)
